# baseline (speedup 1.0000x reference)
import functools

import jax
import jax.numpy as jnp
from jax import lax
from jax.experimental import pallas as pl
from jax.experimental.pallas import tpu as pltpu
from jax.experimental.pallas import tpu_sc as plsc

B = 1024
N = 128
F = 9
H = 64
NW = 32
BPW = B // NW
LG = 16
PG = 4
NCH = (BPW // LG) * (N // PG)
CN = LG * PG


def _sc_body(x_hbm, din_hbm, dout_hbm, atom_hbm, indt_hbm, outdt_hbm, gt_hbm,
             out_hbm,
             xplane, dinp, doutp,
             xidx0, xidx1, didx0, didx1, dodx0, dodx1,
             rows0, rows1, dinr0, dinr1, doutr0, doutr1,
             outs0, outs1, tok, tokslab,
             gsem0, gsem1, wsem0, wsem1):
    wid = lax.axis_index("s") * 2 + lax.axis_index("c")

    xidx = [xidx0, xidx1]
    didx = [didx0, didx1]
    dodx = [dodx0, dodx1]
    rows = [rows0, rows1]
    dinr = [dinr0, dinr1]
    doutr = [doutr0, doutr1]
    outs = [outs0, outs1]
    gsem = [gsem0, gsem1]
    wsem = [wsem0, wsem1]

    iota = jnp.arange(16, dtype=jnp.int32)
    pr = iota >> 2
    pc = iota & 3
    r4c = [iota * 4 + c for c in range(PG)]

    def stage_planes(lg):
        b0 = wid * BPW + lg * LG
        for f in range(F):
            pltpu.sync_copy(x_hbm.at[f, pl.ds(b0, LG), :],
                            xplane.at[pl.ds(f * LG, LG), :])
        pltpu.sync_copy(din_hbm.at[pl.ds(b0, LG), :], dinp)
        pltpu.sync_copy(dout_hbm.at[pl.ds(b0, LG), :], doutp)

    def extract_idx(g, k):
        n0 = lax.rem(g, N // PG) * PG
        colv = pc + n0
        for f in range(F):
            for q in range(CN // 16):
                rowv = pr + (f * LG + q * 4)
                v = plsc.load_gather(xplane, [rowv, colv])
                xidx[k][pl.ds(f * CN + q * 16, 16)] = v
        for q in range(CN // 16):
            rowv = pr + q * 4
            didx[k][pl.ds(q * 16, 16)] = plsc.load_gather(dinp, [rowv, colv])
            dodx[k][pl.ds(q * 16, 16)] = plsc.load_gather(doutp, [rowv, colv])

    def gather_copies(k):
        cps = [
            pltpu.make_async_copy(
                atom_hbm.at[xidx[k].at[pl.ds(f * CN, CN)]],
                rows[k].at[pl.ds(f * CN, CN)], gsem[k])
            for f in range(F)
        ]
        cps.append(pltpu.make_async_copy(indt_hbm.at[didx[k]], dinr[k],
                                         gsem[k]))
        cps.append(pltpu.make_async_copy(outdt_hbm.at[dodx[k]], doutr[k],
                                         gsem[k]))
        return cps

    def wb_copies(g, k):
        n0 = lax.rem(g, N // PG) * PG
        b0 = wid * BPW + (g // (N // PG)) * LG
        return [
            pltpu.make_async_copy(
                outs[k].at[c],
                out_hbm.at[n0 + c + 1, :, pl.ds(b0, LG)], wsem[k])
            for c in range(PG)
        ]

    def reduce_chunk(k):
        def red_body(e, carry):
            colv = jnp.full((16,), 0, jnp.int32) + e
            for c in range(PG):
                vals = [plsc.load_gather(rows[k], [r4c[c] + f * CN, colv])
                        for f in range(F)]
                vals.append(plsc.load_gather(dinr[k], [r4c[c], colv]))
                vals.append(plsc.load_gather(doutr[k], [r4c[c], colv]))
                while len(vals) > 1:
                    nxt = [vals[i] + vals[i + 1]
                           for i in range(0, len(vals) - 1, 2)]
                    if len(vals) % 2:
                        nxt.append(vals[-1])
                    vals = nxt
                outs[k][c, e, :] = vals[0]
            return carry

        lax.fori_loop(0, H, red_body, 0, unroll=2)

    def step(q, epar):
        g = 2 * q + epar
        k = epar
        k1 = 1 - epar
        for cp in gather_copies(k):
            cp.wait()

        @pl.when(g + 1 < NCH)
        def _():
            @pl.when(g + 1 == NCH // 2)
            def _():
                stage_planes(1)

            extract_idx(g + 1, k1)
            for cp in gather_copies(k1):
                cp.start()

        @pl.when(g >= 2)
        def _():
            for cp in wb_copies(g - 2, k):
                cp.wait()

        reduce_chunk(k)
        for cp in wb_copies(g, k):
            cp.start()

    pltpu.sync_copy(gt_hbm, tok)
    for e in range(H):
        ev = jnp.full((16,), e, jnp.int32)
        v = plsc.load_gather(tok, [ev])
        tokslab[e, pl.ds(0, 16)] = v
        tokslab[e, pl.ds(16, 16)] = v
    pltpu.sync_copy(tokslab, out_hbm.at[0, :, pl.ds(wid * BPW, BPW)])

    stage_planes(0)
    extract_idx(0, 0)
    for cp in gather_copies(0):
        cp.start()

    def pair_body(q, carry):
        step(q, 0)
        step(q, 1)
        return carry

    lax.fori_loop(0, NCH // 2, pair_body, 0)

    for cp in wb_copies(NCH - 2, 0):
        cp.wait()
    for cp in wb_copies(NCH - 1, 1):
        cp.wait()


_sc_kernel = functools.partial(
    pl.kernel,
    out_type=jax.ShapeDtypeStruct((N + 1, H, B), jnp.float32),
    mesh=plsc.VectorSubcoreMesh(core_axis_name="c", subcore_axis_name="s"),
    compiler_params=pltpu.CompilerParams(use_tc_tiling_on_sc=False,
                                         needs_layout_passes=False),
    scratch_types=[
        pltpu.VMEM((F * LG, N), jnp.int32),
        pltpu.VMEM((LG, N), jnp.int32),
        pltpu.VMEM((LG, N), jnp.int32),
        pltpu.VMEM((F * CN,), jnp.int32),
        pltpu.VMEM((F * CN,), jnp.int32),
        pltpu.VMEM((CN,), jnp.int32),
        pltpu.VMEM((CN,), jnp.int32),
        pltpu.VMEM((CN,), jnp.int32),
        pltpu.VMEM((CN,), jnp.int32),
        pltpu.VMEM((F * CN, H), jnp.float32),
        pltpu.VMEM((F * CN, H), jnp.float32),
        pltpu.VMEM((CN, H), jnp.float32),
        pltpu.VMEM((CN, H), jnp.float32),
        pltpu.VMEM((CN, H), jnp.float32),
        pltpu.VMEM((CN, H), jnp.float32),
        pltpu.VMEM((PG, H, LG), jnp.float32),
        pltpu.VMEM((PG, H, LG), jnp.float32),
        pltpu.VMEM((H,), jnp.float32),
        pltpu.VMEM((H, BPW), jnp.float32),
        pltpu.SemaphoreType.DMA,
        pltpu.SemaphoreType.DMA,
        pltpu.SemaphoreType.DMA,
        pltpu.SemaphoreType.DMA,
    ],
)(_sc_body)


def kernel(x, in_degree, out_degree, atom_table, in_deg_table, out_deg_table,
           graph_token):
    x_t = jnp.transpose(x.astype(jnp.int32), (2, 0, 1))
    din = in_degree.astype(jnp.int32)
    dout = out_degree.astype(jnp.int32)
    gt = graph_token.reshape(H)
    out = _sc_kernel(x_t, din, dout, atom_table, in_deg_table,
                     out_deg_table, gt)
    return jnp.transpose(out, (2, 0, 1))


# speedup vs baseline: 5.6770x; 5.6770x over previous
import functools

import jax
import jax.numpy as jnp
from jax import lax
from jax.experimental import pallas as pl
from jax.experimental.pallas import tpu as pltpu
from jax.experimental.pallas import tpu_sc as plsc

B = 1024
N = 128
F = 9
H = 64
NW = 32
BPW = B // NW
C = 64
NCH = 2 * BPW
CIDX = C * F


def _sc_body(x_hbm, din_hbm, dout_hbm, atom_hbm, indt_hbm, outdt_hbm, gt_hbm,
             out_hbm,
             xidx0, xidx1, din0, din1, dout0, dout1,
             rows0, rows1, dinr0, dinr1, doutr0, doutr1,
             outb0, outb1, gt_v,
             isem0, isem1, gsem0, gsem1, wsem0, wsem1):
    wid = lax.axis_index("s") * 2 + lax.axis_index("c")

    xidx = [xidx0, xidx1]
    din = [din0, din1]
    dout = [dout0, dout1]
    rows = [rows0, rows1]
    dinr = [dinr0, dinr1]
    doutr = [doutr0, doutr1]
    outb = [outb0, outb1]
    isem = [isem0, isem1]
    gsem = [gsem0, gsem1]
    wsem = [wsem0, wsem1]

    pltpu.sync_copy(gt_hbm, gt_v)
    for j in range(H // 16):
        outb0[0, pl.ds(j * 16, 16)] = gt_v[pl.ds(j * 16, 16)]

    def idx_copies(g, k):
        b = wid * BPW + g // 2
        h = g % 2
        cps = [
            pltpu.make_async_copy(
                x_hbm.at[f, b, pl.ds(h * C, C)], xidx[k].at[f], isem[k])
            for f in range(F)
        ]
        cps.append(pltpu.make_async_copy(
            din_hbm.at[pl.ds(wid * (BPW * N) + g * C, C)], din[k], isem[k]))
        cps.append(pltpu.make_async_copy(
            dout_hbm.at[pl.ds(wid * (BPW * N) + g * C, C)], dout[k], isem[k]))
        return cps

    def gather_copies(k):
        cps = [
            pltpu.make_async_copy(
                atom_hbm.at[xidx[k].at[f]],
                rows[k].at[pl.ds(f * C, C)], gsem[k])
            for f in range(F)
        ]
        cps.append(pltpu.make_async_copy(indt_hbm.at[din[k]], dinr[k],
                                         gsem[k]))
        cps.append(pltpu.make_async_copy(outdt_hbm.at[dout[k]], doutr[k],
                                         gsem[k]))
        return cps

    def wb_copy(q, e):
        b = wid * BPW + q
        if e == 0:
            dst = out_hbm.at[b, pl.ds(0, C + 1)]
        else:
            dst = out_hbm.at[b, pl.ds(C + 1, C)]
        return pltpu.make_async_copy(outb[e], dst, wsem[e])

    def step(q, e):
        g = 2 * q + e
        k = e
        k1 = 1 - e
        for c in gather_copies(k):
            c.wait()

        @pl.when(g + 2 < NCH)
        def _():
            for c in idx_copies(g + 2, k):
                c.start()

        @pl.when(g + 1 < NCH)
        def _():
            for c in idx_copies(g + 1, k1):
                c.wait()
            for c in gather_copies(k1):
                c.start()

        @pl.when(g >= 2)
        def _():
            wb_copy(q - 1, e).wait()

        base_row = 1 if e == 0 else 0

        def node_body(n, c2):
            for j in range(H // 16):
                sl = pl.ds(j * 16, 16)
                vals = [rows[k][f * C + n, sl] for f in range(F)]
                vals.append(dinr[k][n, sl])
                vals.append(doutr[k][n, sl])
                while len(vals) > 1:
                    nxt = [vals[i] + vals[i + 1]
                           for i in range(0, len(vals) - 1, 2)]
                    if len(vals) % 2:
                        nxt.append(vals[-1])
                    vals = nxt
                outb[k][base_row + n, sl] = vals[0]
            return c2

        lax.fori_loop(0, C, node_body, 0, unroll=4)
        wb_copy(q, e).start()

    for c in idx_copies(0, 0):
        c.start()
    for c in idx_copies(1, 1):
        c.start()
    for c in idx_copies(0, 0):
        c.wait()
    for c in gather_copies(0):
        c.start()

    def pair_body(q, carry):
        step(q, 0)
        step(q, 1)
        return carry

    lax.fori_loop(0, BPW, pair_body, 0)

    wb_copy(BPW - 1, 0).wait()
    wb_copy(BPW - 1, 1).wait()


_sc_kernel = functools.partial(
    pl.kernel,
    out_type=jax.ShapeDtypeStruct((B, N + 1, H), jnp.float32),
    mesh=plsc.VectorSubcoreMesh(core_axis_name="c", subcore_axis_name="s"),
    compiler_params=pltpu.CompilerParams(use_tc_tiling_on_sc=False),
    scratch_types=[
        pltpu.VMEM((F, C), jnp.int32),
        pltpu.VMEM((F, C), jnp.int32),
        pltpu.VMEM((C,), jnp.int32),
        pltpu.VMEM((C,), jnp.int32),
        pltpu.VMEM((C,), jnp.int32),
        pltpu.VMEM((C,), jnp.int32),
        pltpu.VMEM((CIDX, H), jnp.float32),
        pltpu.VMEM((CIDX, H), jnp.float32),
        pltpu.VMEM((C, H), jnp.float32),
        pltpu.VMEM((C, H), jnp.float32),
        pltpu.VMEM((C, H), jnp.float32),
        pltpu.VMEM((C, H), jnp.float32),
        pltpu.VMEM((C + 1, H), jnp.float32),
        pltpu.VMEM((C, H), jnp.float32),
        pltpu.VMEM((H,), jnp.float32),
        pltpu.SemaphoreType.DMA,
        pltpu.SemaphoreType.DMA,
        pltpu.SemaphoreType.DMA,
        pltpu.SemaphoreType.DMA,
        pltpu.SemaphoreType.DMA,
        pltpu.SemaphoreType.DMA,
    ],
)(_sc_body)


def kernel(x, in_degree, out_degree, atom_table, in_deg_table, out_deg_table,
           graph_token):
    x_t = jnp.transpose(x.astype(jnp.int32), (2, 0, 1))
    din = in_degree.reshape(-1).astype(jnp.int32)
    dout = out_degree.reshape(-1).astype(jnp.int32)
    gt = graph_token.reshape(H)
    return _sc_kernel(x_t, din, dout, atom_table, in_deg_table,
                      out_deg_table, gt)


# speedup vs baseline: 5.6784x; 1.0002x over previous
import functools

import jax
import jax.numpy as jnp
from jax import lax
from jax.experimental import pallas as pl
from jax.experimental.pallas import tpu as pltpu
from jax.experimental.pallas import tpu_sc as plsc

B = 1024
N = 128
F = 9
H = 64
NW = 32
BPW = B // NW
C = 64
NCH = 2 * BPW
CIDX = C * F


def _sc_body(x_hbm, din_hbm, dout_hbm, atom_hbm, indt_hbm, outdt_hbm, gt_hbm,
             out_hbm,
             xidx0, xidx1, din0, din1, dout0, dout1,
             rows0, rows1, dinr0, dinr1, doutr0, doutr1,
             outb0, outb1, gt_v,
             isem0, isem1, gsem0, gsem1, wsem0, wsem1):
    wid = lax.axis_index("s") * 2 + lax.axis_index("c")

    xidx = [xidx0, xidx1]
    din = [din0, din1]
    dout = [dout0, dout1]
    rows = [rows0, rows1]
    dinr = [dinr0, dinr1]
    doutr = [doutr0, doutr1]
    outb = [outb0, outb1]
    isem = [isem0, isem1]
    gsem = [gsem0, gsem1]
    wsem = [wsem0, wsem1]

    pltpu.sync_copy(gt_hbm, gt_v)
    for j in range(H // 16):
        outb0[0, pl.ds(j * 16, 16)] = gt_v[pl.ds(j * 16, 16)]

    def idx_copies(g, k):
        b = wid * BPW + g // 2
        h = g % 2
        cps = [
            pltpu.make_async_copy(
                x_hbm.at[f, b, pl.ds(h * C, C)], xidx[k].at[f], isem[k])
            for f in range(F)
        ]
        cps.append(pltpu.make_async_copy(
            din_hbm.at[pl.ds(wid * (BPW * N) + g * C, C)], din[k], isem[k]))
        cps.append(pltpu.make_async_copy(
            dout_hbm.at[pl.ds(wid * (BPW * N) + g * C, C)], dout[k], isem[k]))
        return cps

    def gather_copies(k):
        cps = [
            pltpu.make_async_copy(
                atom_hbm.at[xidx[k].at[f]],
                rows[k].at[pl.ds(f * C, C)], gsem[k])
            for f in range(F)
        ]
        cps.append(pltpu.make_async_copy(indt_hbm.at[din[k]], dinr[k],
                                         gsem[k]))
        cps.append(pltpu.make_async_copy(outdt_hbm.at[dout[k]], doutr[k],
                                         gsem[k]))
        return cps

    def wb_copy(q, e):
        b = wid * BPW + q
        if e == 0:
            dst = out_hbm.at[b, pl.ds(0, C + 1)]
        else:
            dst = out_hbm.at[b, pl.ds(C + 1, C)]
        return pltpu.make_async_copy(outb[e], dst, wsem[e])

    def step(q, e):
        g = 2 * q + e
        k = e
        k1 = 1 - e
        for c in gather_copies(k):
            c.wait()

        @pl.when(g + 2 < NCH)
        def _():
            for c in idx_copies(g + 2, k):
                c.start()

        @pl.when(g + 1 < NCH)
        def _():
            for c in idx_copies(g + 1, k1):
                c.wait()
            for c in gather_copies(k1):
                c.start()

        @pl.when(g >= 2)
        def _():
            wb_copy(q - 1, e).wait()

        base_row = 1 if e == 0 else 0

        def node_body(n, c2):
            for j in range(H // 16):
                sl = pl.ds(j * 16, 16)
                vals = [rows[k][f * C + n, sl] for f in range(F)]
                vals.append(dinr[k][n, sl])
                vals.append(doutr[k][n, sl])
                while len(vals) > 1:
                    nxt = [vals[i] + vals[i + 1]
                           for i in range(0, len(vals) - 1, 2)]
                    if len(vals) % 2:
                        nxt.append(vals[-1])
                    vals = nxt
                outb[k][base_row + n, sl] = vals[0]
            return c2

        lax.fori_loop(0, C, node_body, 0, unroll=2)
        wb_copy(q, e).start()

    for c in idx_copies(0, 0):
        c.start()
    for c in idx_copies(1, 1):
        c.start()
    for c in idx_copies(0, 0):
        c.wait()
    for c in gather_copies(0):
        c.start()

    def pair_body(q, carry):
        step(q, 0)
        step(q, 1)
        return carry

    lax.fori_loop(0, BPW, pair_body, 0)

    wb_copy(BPW - 1, 0).wait()
    wb_copy(BPW - 1, 1).wait()


_sc_kernel = functools.partial(
    pl.kernel,
    out_type=jax.ShapeDtypeStruct((B, N + 1, H), jnp.float32),
    mesh=plsc.VectorSubcoreMesh(core_axis_name="c", subcore_axis_name="s"),
    compiler_params=pltpu.CompilerParams(use_tc_tiling_on_sc=False),
    scratch_types=[
        pltpu.VMEM((F, C), jnp.int32),
        pltpu.VMEM((F, C), jnp.int32),
        pltpu.VMEM((C,), jnp.int32),
        pltpu.VMEM((C,), jnp.int32),
        pltpu.VMEM((C,), jnp.int32),
        pltpu.VMEM((C,), jnp.int32),
        pltpu.VMEM((CIDX, H), jnp.float32),
        pltpu.VMEM((CIDX, H), jnp.float32),
        pltpu.VMEM((C, H), jnp.float32),
        pltpu.VMEM((C, H), jnp.float32),
        pltpu.VMEM((C, H), jnp.float32),
        pltpu.VMEM((C, H), jnp.float32),
        pltpu.VMEM((C + 1, H), jnp.float32),
        pltpu.VMEM((C, H), jnp.float32),
        pltpu.VMEM((H,), jnp.float32),
        pltpu.SemaphoreType.DMA,
        pltpu.SemaphoreType.DMA,
        pltpu.SemaphoreType.DMA,
        pltpu.SemaphoreType.DMA,
        pltpu.SemaphoreType.DMA,
        pltpu.SemaphoreType.DMA,
    ],
)(_sc_body)


def kernel(x, in_degree, out_degree, atom_table, in_deg_table, out_deg_table,
           graph_token):
    x_t = jnp.transpose(x.astype(jnp.int32), (2, 0, 1))
    din = in_degree.reshape(-1).astype(jnp.int32)
    dout = out_degree.reshape(-1).astype(jnp.int32)
    gt = graph_token.reshape(H)
    return _sc_kernel(x_t, din, dout, atom_table, in_deg_table,
                      out_deg_table, gt)
